# baseline (speedup 1.0000x reference)
import functools

import jax
import jax.numpy as jnp
from jax import lax
from jax.experimental import pallas as pl
from jax.experimental.pallas import tpu as pltpu
from jax.experimental.pallas import tpu_sc as plsc

N = 4096
K = 20
BN = 256
_HIGH = lax.Precision.HIGHEST


def _dot(a, b):
    return lax.dot_general(a, b, (((1,), (0,)), ((), ())),
                           precision=_HIGH, preferred_element_type=jnp.float32)


def _edge_body(x_ref, w_ref, b_ref, q_ref, pb_ref, idx_ref, *, F, H):
    i = pl.program_id(0)
    x_all = x_ref[...]
    xb = x_ref[pl.ds(i * BN, BN), :]
    wa = w_ref[0:F, :]
    wb = w_ref[F : 2 * F, :]

    q_ref[...] = _dot(xb, wb)
    pb_ref[...] = _dot(xb, wa - wb) + b_ref[...]

    inner = lax.dot_general(xb.astype(jnp.bfloat16), x_all.astype(jnp.bfloat16),
                            (((1,), (1,)), ((), ())),
                            preferred_element_type=jnp.float32)
    sq_all = jnp.sum(x_all * x_all, axis=1)[None, :]
    sqb = jnp.sum(xb * xb, axis=1, keepdims=True)
    d2 = sqb - 2.0 * inner + sq_all

    colvf = lax.broadcasted_iota(jnp.int32, (BN, N), 1).astype(jnp.float32)
    kcol = lax.broadcasted_iota(jnp.int32, (BN, K), 1)
    idx_acc = jnp.zeros((BN, K), jnp.int32)
    for t in range(K):
        v, ix = d2, colvf
        w = N
        while w > 1:
            w //= 2
            keep = v[:, :w] <= v[:, w:2 * w]
            v = jnp.where(keep, v[:, :w], v[:, w:2 * w])
            ix = jnp.where(keep, ix[:, :w], ix[:, w:2 * w])
        aminf = ix
        idx_acc = jnp.where(kcol == t, aminf.astype(jnp.int32), idx_acc)
        d2 = jnp.where(colvf == aminf, jnp.inf, d2)
    idx_ref[...] = idx_acc


def _edge_call(x, w, b2d, F, H):
    grid = N // BN
    body = functools.partial(_edge_body, F=F, H=H)
    return pl.pallas_call(
        body,
        grid=(grid,),
        in_specs=[
            pl.BlockSpec((N, F), lambda i: (0, 0)),
            pl.BlockSpec((2 * F, H), lambda i: (0, 0)),
            pl.BlockSpec((1, H), lambda i: (0, 0)),
        ],
        out_specs=[
            pl.BlockSpec((BN, H), lambda i: (i, 0)),
            pl.BlockSpec((BN, H), lambda i: (i, 0)),
            pl.BlockSpec((BN, K), lambda i: (i, 0)),
        ],
        out_shape=[
            jax.ShapeDtypeStruct((N, H), jnp.float32),
            jax.ShapeDtypeStruct((N, H), jnp.float32),
            jax.ShapeDtypeStruct((N, K), jnp.int32),
        ],
        compiler_params=pltpu.CompilerParams(
            dimension_semantics=("arbitrary",)),
    )(x, w, b2d)


_NC, _NS = 2, 16
_NW = _NC * _NS


@functools.lru_cache(maxsize=None)
def _make_gather_max(H):
    rpw = N // _NW
    rchunk = 4
    nidx = rchunk * K
    nch = rpw // rchunk
    nbuf = 4
    mesh = plsc.VectorSubcoreMesh(core_axis_name="c", subcore_axis_name="s")

    @functools.partial(
        pl.kernel,
        mesh=mesh,
        out_type=jax.ShapeDtypeStruct((N, H), jnp.float32),
        scratch_types=[
            pltpu.VMEM((rpw * K,), jnp.int32),
        ] + [pltpu.VMEM((nidx, H), jnp.float32) for _ in range(nbuf)] + [
            pltpu.VMEM((rchunk, H), jnp.float32),
            pltpu.VMEM((rchunk, H), jnp.float32),
        ] + [pltpu.SemaphoreType.DMA for _ in range(nbuf)],
    )
    def k(q_hbm, pb_hbm, idx_hbm, out_hbm, idx_v, *rest):
        bufs, (pbv, outv), sems = rest[:nbuf], rest[nbuf:nbuf + 2], rest[nbuf + 2:]
        wid = lax.axis_index("s") * _NC + lax.axis_index("c")
        base_row = wid * rpw
        pltpu.sync_copy(idx_hbm.at[pl.ds(base_row * K, rpw * K)], idx_v)

        def fire(c, b):
            pltpu.async_copy(
                q_hbm.at[idx_v.at[pl.ds(c * nidx, nidx)]], bufs[b], sems[b])

        for b in range(nbuf):
            fire(b, b)

        def ring_body(i, carry):
            for b in range(nbuf):
                c = nbuf * i + b
                pltpu.make_async_copy(
                    q_hbm.at[pl.ds(0, nidx), :], bufs[b], sems[b]).wait()
                row0 = base_row + c * rchunk
                pltpu.sync_copy(pb_hbm.at[pl.ds(row0, rchunk), :], pbv)

                def gbody(g, _, b=b):
                    sl = pl.ds(g * 16, 16)
                    for r in range(rchunk):
                        vs = [bufs[b][r * K + j, sl] for j in range(K)]
                        while len(vs) > 1:
                            vs = [jnp.maximum(vs[t], vs[t + 1])
                                  for t in range(0, len(vs) - 1, 2)] + (
                                      [vs[-1]] if len(vs) % 2 else [])
                        outv[r, sl] = jnp.maximum(pbv[r, sl] + vs[0], 0.0)
                    return 0

                lax.fori_loop(0, H // 16, gbody, 0)
                pltpu.sync_copy(outv, out_hbm.at[pl.ds(row0, rchunk), :])

                @pl.when(c + nbuf < nch)
                def _():
                    fire(c + nbuf, b)

            return carry

        lax.fori_loop(0, nch // nbuf, ring_body, 0)

    return k


def _head_body(h_ref, w3_ref, b3_ref, w4_ref, b4_ref, o_ref):
    t = jnp.maximum(_dot(h_ref[...], w3_ref[...]) + b3_ref[...], 0.0)
    o_ref[...] = jax.nn.sigmoid(_dot(t, w4_ref[...]) + b4_ref[...])


def _head_call(h, w3, b3, w4, b4):
    H2, H3, OUT = w3.shape[0], w3.shape[1], w4.shape[1]
    bn = 512
    return pl.pallas_call(
        _head_body,
        grid=(N // bn,),
        in_specs=[
            pl.BlockSpec((bn, H2), lambda i: (i, 0)),
            pl.BlockSpec((H2, H3), lambda i: (0, 0)),
            pl.BlockSpec((1, H3), lambda i: (0, 0)),
            pl.BlockSpec((H3, OUT), lambda i: (0, 0)),
            pl.BlockSpec((1, OUT), lambda i: (0, 0)),
        ],
        out_specs=pl.BlockSpec((bn, OUT), lambda i: (i, 0)),
        out_shape=jax.ShapeDtypeStruct((N, OUT), jnp.float32),
        compiler_params=pltpu.CompilerParams(
            dimension_semantics=("arbitrary",)),
    )(h, w3, b3, w4, b4)


def _gather_max(q, pb, idxf, H):
    return _make_gather_max(H)(q, pb, idxf)


def _layer(x, w, b, F, H):
    q, pb, idx = _edge_call(x, w, b.reshape(1, H), F, H)
    return _gather_max(q, pb, idx.reshape(N * K), H)


def kernel(x, W1, b1, W2, b2, W3, b3, W4, b4):
    h1 = _layer(x, W1, b1, 128, 128)
    h2 = _layer(h1, W2, b2, 128, 256)
    return _head_call(h2, W3, b3.reshape(1, -1), W4, b4.reshape(1, -1))


# speedup vs baseline: 1.4667x; 1.4667x over previous
import functools

import jax
import jax.numpy as jnp
from jax import lax
from jax.experimental import pallas as pl
from jax.experimental.pallas import tpu as pltpu
from jax.experimental.pallas import tpu_sc as plsc

N = 4096
K = 20
BN = 256
_HIGH = lax.Precision.HIGHEST


def _dot(a, b):
    return lax.dot_general(a, b, (((1,), (0,)), ((), ())),
                           precision=_HIGH, preferred_element_type=jnp.float32)


def _edge_body(x_ref, w_ref, b_ref, q_ref, pb_ref, idx_ref, *, F, H):
    i = pl.program_id(0)
    x_all = x_ref[...]
    xb = x_ref[pl.ds(i * BN, BN), :]
    wa = w_ref[0:F, :]
    wb = w_ref[F : 2 * F, :]

    q_ref[...] = _dot(xb, wb)
    pb_ref[...] = _dot(xb, wa - wb) + b_ref[...]

    inner = lax.dot_general(xb.astype(jnp.bfloat16), x_all.astype(jnp.bfloat16),
                            (((1,), (1,)), ((), ())),
                            preferred_element_type=jnp.float32)
    sq_all = jnp.sum(x_all * x_all, axis=1)[None, :]
    sqb = jnp.sum(xb * xb, axis=1, keepdims=True)
    d2 = sqb - 2.0 * inner + sq_all

    colvf = lax.broadcasted_iota(jnp.int32, (BN, N), 1).astype(jnp.float32)
    kcol = lax.broadcasted_iota(jnp.int32, (BN, K), 1)
    idx_acc = jnp.zeros((BN, K), jnp.int32)
    for t in range(K):
        m = jnp.min(d2, axis=1, keepdims=True)
        aminf = jnp.min(jnp.where(d2 == m, colvf, jnp.inf), axis=1,
                        keepdims=True)
        idx_acc = jnp.where(kcol == t, aminf.astype(jnp.int32), idx_acc)
        d2 = jnp.where(colvf == aminf, jnp.inf, d2)
    idx_ref[...] = idx_acc


def _edge_call(x, w, b2d, F, H):
    grid = N // BN
    body = functools.partial(_edge_body, F=F, H=H)
    return pl.pallas_call(
        body,
        grid=(grid,),
        in_specs=[
            pl.BlockSpec((N, F), lambda i: (0, 0)),
            pl.BlockSpec((2 * F, H), lambda i: (0, 0)),
            pl.BlockSpec((1, H), lambda i: (0, 0)),
        ],
        out_specs=[
            pl.BlockSpec((BN, H), lambda i: (i, 0)),
            pl.BlockSpec((BN, H), lambda i: (i, 0)),
            pl.BlockSpec((BN, K), lambda i: (i, 0)),
        ],
        out_shape=[
            jax.ShapeDtypeStruct((N, H), jnp.float32),
            jax.ShapeDtypeStruct((N, H), jnp.float32),
            jax.ShapeDtypeStruct((N, K), jnp.int32),
        ],
        compiler_params=pltpu.CompilerParams(
            dimension_semantics=("arbitrary",)),
    )(x, w, b2d)


_NC, _NS = 2, 16
_NW = _NC * _NS


@functools.lru_cache(maxsize=None)
def _make_gather_max(H):
    rpw = N // _NW
    rchunk = 4
    nidx = rchunk * K
    nch = rpw // rchunk
    nbuf = 4
    mesh = plsc.VectorSubcoreMesh(core_axis_name="c", subcore_axis_name="s")

    @functools.partial(
        pl.kernel,
        mesh=mesh,
        out_type=jax.ShapeDtypeStruct((N, H), jnp.float32),
        scratch_types=[
            pltpu.VMEM((rpw * K,), jnp.int32),
        ] + [pltpu.VMEM((nidx, H), jnp.float32) for _ in range(nbuf)] + [
            pltpu.VMEM((rchunk, H), jnp.float32),
            pltpu.VMEM((rchunk, H), jnp.float32),
        ] + [pltpu.SemaphoreType.DMA for _ in range(nbuf)],
    )
    def k(q_hbm, pb_hbm, idx_hbm, out_hbm, idx_v, *rest):
        bufs, (pbv, outv), sems = rest[:nbuf], rest[nbuf:nbuf + 2], rest[nbuf + 2:]
        wid = lax.axis_index("s") * _NC + lax.axis_index("c")
        base_row = wid * rpw
        pltpu.sync_copy(idx_hbm.at[pl.ds(base_row * K, rpw * K)], idx_v)

        def fire(c, b):
            pltpu.async_copy(
                q_hbm.at[idx_v.at[pl.ds(c * nidx, nidx)]], bufs[b], sems[b])

        for b in range(nbuf):
            fire(b, b)

        def ring_body(i, carry):
            for b in range(nbuf):
                c = nbuf * i + b
                pltpu.make_async_copy(
                    q_hbm.at[pl.ds(0, nidx), :], bufs[b], sems[b]).wait()
                row0 = base_row + c * rchunk
                pltpu.sync_copy(pb_hbm.at[pl.ds(row0, rchunk), :], pbv)

                def gbody(g, _, b=b):
                    sl = pl.ds(g * 16, 16)
                    for r in range(rchunk):
                        vs = [bufs[b][r * K + j, sl] for j in range(K)]
                        while len(vs) > 1:
                            vs = [jnp.maximum(vs[t], vs[t + 1])
                                  for t in range(0, len(vs) - 1, 2)] + (
                                      [vs[-1]] if len(vs) % 2 else [])
                        outv[r, sl] = jnp.maximum(pbv[r, sl] + vs[0], 0.0)
                    return 0

                lax.fori_loop(0, H // 16, gbody, 0)
                pltpu.sync_copy(outv, out_hbm.at[pl.ds(row0, rchunk), :])

                @pl.when(c + nbuf < nch)
                def _():
                    fire(c + nbuf, b)

            return carry

        lax.fori_loop(0, nch // nbuf, ring_body, 0)

    return k


def _head_body(h_ref, w3_ref, b3_ref, w4_ref, b4_ref, o_ref):
    t = jnp.maximum(_dot(h_ref[...], w3_ref[...]) + b3_ref[...], 0.0)
    o_ref[...] = jax.nn.sigmoid(_dot(t, w4_ref[...]) + b4_ref[...])


def _head_call(h, w3, b3, w4, b4):
    H2, H3, OUT = w3.shape[0], w3.shape[1], w4.shape[1]
    bn = 512
    return pl.pallas_call(
        _head_body,
        grid=(N // bn,),
        in_specs=[
            pl.BlockSpec((bn, H2), lambda i: (i, 0)),
            pl.BlockSpec((H2, H3), lambda i: (0, 0)),
            pl.BlockSpec((1, H3), lambda i: (0, 0)),
            pl.BlockSpec((H3, OUT), lambda i: (0, 0)),
            pl.BlockSpec((1, OUT), lambda i: (0, 0)),
        ],
        out_specs=pl.BlockSpec((bn, OUT), lambda i: (i, 0)),
        out_shape=jax.ShapeDtypeStruct((N, OUT), jnp.float32),
        compiler_params=pltpu.CompilerParams(
            dimension_semantics=("arbitrary",)),
    )(h, w3, b3, w4, b4)


def _gather_max(q, pb, idxf, H):
    return _make_gather_max(H)(q, pb, idxf)


def _layer(x, w, b, F, H):
    q, pb, idx = _edge_call(x, w, b.reshape(1, H), F, H)
    return _gather_max(q, pb, idx.reshape(N * K), H)


def kernel(x, W1, b1, W2, b2, W3, b3, W4, b4):
    h1 = _layer(x, W1, b1, 128, 128)
    h2 = _layer(h1, W2, b2, 128, 256)
    return _head_call(h2, W3, b3.reshape(1, -1), W4, b4.reshape(1, -1))
